# baseline (speedup 1.0000x reference)
import jax
import jax.numpy as jnp
from jax import lax
from jax.experimental import pallas as pl
from jax.experimental.pallas import tpu as pltpu
from jax.experimental.pallas import tpu_sc as plsc

NUM_STYLES = 100000
EMBED_DIM = 64
BATCH = 16384

_info = plsc.get_sparse_core_info()
_NC, _NS = _info.num_cores, _info.num_subcores
_NW = _NC * _NS
_BPW = BATCH // _NW
_CH = 128
_NCH = _BPW // _CH


def _gather_body(w3_hbm, idx_hbm, out_hbm, idx_v, rows_v, sem_a, sem_b, sem_w):
    wid = lax.axis_index("s") * _NC + lax.axis_index("c")
    base = wid * _BPW
    pltpu.sync_copy(idx_hbm.at[pl.ds(base, _BPW)], idx_v)
    sems = [sem_a, sem_b]

    def drain_and_write(p):
        pltpu.make_async_copy(
            out_hbm.at[pl.ds(base + p * _CH, _CH)],
            rows_v.at[pl.ds(p * _CH, _CH)],
            sems[p % 2],
        ).wait()
        pltpu.async_copy(
            rows_v.at[pl.ds(p * _CH, _CH)],
            out_hbm.at[pl.ds(base + p * _CH, _CH)],
            sem_w,
        )

    for c in range(_NCH):

        def group_body(k, carry, c=c):
            g0 = c * _CH + k * 16
            vg = idx_v[pl.ds(g0, 16)]
            bv = lax.shift_right_logical(vg, 3)
            rv = vg & 7
            for t in range(16):
                pltpu.async_copy(
                    w3_hbm.at[bv[t], rv[t]], rows_v.at[g0 + t], sems[c % 2]
                )
            return carry

        lax.fori_loop(0, _CH // 16, group_body, 0)
        if c >= 1:
            drain_and_write(c - 1)

    drain_and_write(_NCH - 1)
    pltpu.make_async_copy(
        out_hbm.at[pl.ds(base, _BPW)], rows_v, sem_w
    ).wait()


_gather = pl.kernel(
    _gather_body,
    mesh=plsc.VectorSubcoreMesh(core_axis_name="c", subcore_axis_name="s"),
    out_type=jax.ShapeDtypeStruct((BATCH, EMBED_DIM), jnp.float32),
    scratch_types=[
        pltpu.VMEM((_BPW,), jnp.int32),
        pltpu.VMEM((_BPW, EMBED_DIM), jnp.float32),
        pltpu.SemaphoreType.DMA,
        pltpu.SemaphoreType.DMA,
        pltpu.SemaphoreType.DMA,
    ],
)


@jax.jit
def kernel(style_idx, weight):
    w3 = weight.reshape(NUM_STYLES // 8, 8, EMBED_DIM)
    return _gather(w3, style_idx.astype(jnp.int32))


# speedup vs baseline: 1.0118x; 1.0118x over previous
import jax
import jax.numpy as jnp
from jax import lax
from jax.experimental import pallas as pl
from jax.experimental.pallas import tpu as pltpu
from jax.experimental.pallas import tpu_sc as plsc

NUM_STYLES = 100000
EMBED_DIM = 64
BATCH = 16384

_info = plsc.get_sparse_core_info()
_NC, _NS = _info.num_cores, _info.num_subcores
_NW = _NC * _NS
_BPW = BATCH // _NW


def _gather_body(w3_hbm, idx_hbm, out_hbm, idx_v, rows_v, sem):
    wid = lax.axis_index("s") * _NC + lax.axis_index("c")
    base = wid * _BPW
    pltpu.sync_copy(idx_hbm.at[pl.ds(base, _BPW)], idx_v)

    def group_body(k, carry):
        g0 = k * 16
        vg = idx_v[pl.ds(g0, 16)]
        bv = lax.shift_right_logical(vg, 3)
        rv = vg & 7
        for t in range(16):
            pltpu.async_copy(
                w3_hbm.at[bv[t], rv[t]], rows_v.at[g0 + t], sem
            )
        return carry

    lax.fori_loop(0, _BPW // 16, group_body, 0)
    pltpu.make_async_copy(out_hbm.at[pl.ds(base, _BPW)], rows_v, sem).wait()
    pltpu.sync_copy(rows_v, out_hbm.at[pl.ds(base, _BPW)])


_gather = pl.kernel(
    _gather_body,
    mesh=plsc.VectorSubcoreMesh(core_axis_name="c", subcore_axis_name="s"),
    out_type=jax.ShapeDtypeStruct((BATCH, EMBED_DIM), jnp.float32),
    scratch_types=[
        pltpu.VMEM((_BPW,), jnp.int32),
        pltpu.VMEM((_BPW, EMBED_DIM), jnp.float32),
        pltpu.SemaphoreType.DMA,
    ],
)


@jax.jit
def kernel(style_idx, weight):
    w3 = weight.reshape(NUM_STYLES // 8, 8, EMBED_DIM)
    return _gather(w3, style_idx.astype(jnp.int32))
